# Initial kernel scaffold; baseline (speedup 1.0000x reference)
import jax
import jax.numpy as jnp
from jax.experimental import pallas as pl


def kernel(x, w):
    raise NotImplementedError("write your pallas kernel here")


# speedup vs baseline: 1.9672x; 1.9672x over previous
import jax
import jax.numpy as jnp
from jax.experimental import pallas as pl
from jax.experimental.pallas import tpu as pltpu

_EMB = 64
_NEMB = 100
_BLK = 2048


def _body(x_ref, w_ref, wt_ref, c_ref, w2_ref, idx_ref, q_ref):
    xb = x_ref[...]
    wm = w_ref[...]
    xw = jax.lax.dot_general(
        xb, wm, dimension_numbers=(((1,), (0,)), ((), ())),
        preferred_element_type=jnp.float32)
    d = c_ref[...] - 2.0 * xw + w2_ref[...]
    neg = -d
    m = jnp.max(neg, axis=1, keepdims=True)
    cols = jax.lax.broadcasted_iota(jnp.int32, neg.shape, 1)
    idx = jnp.min(jnp.where(neg == m, cols, _NEMB), axis=1)
    idx_ref[0, 0, :] = idx
    onehot = (cols == idx[:, None]).astype(jnp.float32)
    q_ref[...] = jax.lax.dot_general(
        onehot, wt_ref[...], dimension_numbers=(((1,), (0,)), ((), ())),
        precision=jax.lax.Precision.HIGHEST,
        preferred_element_type=jnp.float32)


def kernel(x, w):
    flat = x.reshape(-1, _EMB)
    tokens = flat.shape[0]
    nb = tokens // _BLK
    wt = w.T
    c = jnp.sum(flat ** 2, axis=1, keepdims=True)
    w2 = jnp.sum(w ** 2, axis=0, keepdims=True)
    idx, q = pl.pallas_call(
        _body,
        grid=(nb,),
        in_specs=[
            pl.BlockSpec((_BLK, _EMB), lambda i: (i, 0)),
            pl.BlockSpec((_EMB, _NEMB), lambda i: (0, 0)),
            pl.BlockSpec((_NEMB, _EMB), lambda i: (0, 0)),
            pl.BlockSpec((_BLK, 1), lambda i: (i, 0)),
            pl.BlockSpec((1, _NEMB), lambda i: (0, 0)),
        ],
        out_specs=[
            pl.BlockSpec((1, 1, _BLK), lambda i: (i, 0, 0)),
            pl.BlockSpec((_BLK, _EMB), lambda i: (i, 0)),
        ],
        out_shape=[
            jax.ShapeDtypeStruct((nb, 1, _BLK), jnp.int32),
            jax.ShapeDtypeStruct((tokens, _EMB), jnp.float32),
        ],
        compiler_params=pltpu.CompilerParams(
            dimension_semantics=("parallel",)),
    )(flat, w, wt, c, w2)
    return q.reshape(x.shape), idx.reshape(x.shape[:-1])


# speedup vs baseline: 2.1332x; 1.0844x over previous
import jax
import jax.numpy as jnp
from jax.experimental import pallas as pl
from jax.experimental.pallas import tpu as pltpu

_EMB = 64
_NEMB = 100
_BLK = 2048


def _body(x_ref, w2x_ref, wt1_ref, wt2_ref, wt3_ref, c_ref, w2_ref,
          idx_ref, q_ref):
    xb = x_ref[...]
    xw2 = jax.lax.dot_general(
        xb, w2x_ref[...], dimension_numbers=(((1,), (0,)), ((), ())),
        preferred_element_type=jnp.float32)
    neg = (xw2 - c_ref[...]) - w2_ref[...]
    m = jnp.max(neg, axis=1, keepdims=True)
    colsf = jax.lax.broadcasted_iota(
        jnp.int32, neg.shape, 1).astype(jnp.float32)
    idxf = jnp.min(jnp.where(neg == m, colsf, jnp.float32(_NEMB)),
                   axis=1, keepdims=True)
    idx_ref[...] = idxf.astype(jnp.int32)
    onehot = (colsf == idxf).astype(jnp.bfloat16)
    dn = (((1,), (0,)), ((), ()))
    q_ref[...] = (
        (jax.lax.dot_general(onehot, wt1_ref[...], dn,
                             preferred_element_type=jnp.float32)
         + jax.lax.dot_general(onehot, wt2_ref[...], dn,
                               preferred_element_type=jnp.float32))
        + jax.lax.dot_general(onehot, wt3_ref[...], dn,
                              preferred_element_type=jnp.float32))


def kernel(x, w):
    flat = x.reshape(-1, _EMB)
    tokens = flat.shape[0]
    nb = tokens // _BLK
    w2x = 2.0 * w
    wt = w.T
    def _tr(v):
        b = jax.lax.bitcast_convert_type(v, jnp.uint32)
        return jax.lax.bitcast_convert_type(b & jnp.uint32(0xFFFF0000),
                                            jnp.float32)
    p1 = _tr(wt)
    r1 = wt - p1
    p2 = _tr(r1)
    p3 = r1 - p2
    wt1 = p1.astype(jnp.bfloat16)
    wt2 = p2.astype(jnp.bfloat16)
    wt3 = p3.astype(jnp.bfloat16)
    c = jnp.sum(flat ** 2, axis=1, keepdims=True)
    w2 = jnp.sum(w ** 2, axis=0, keepdims=True)
    idx, q = pl.pallas_call(
        _body,
        grid=(nb,),
        in_specs=[
            pl.BlockSpec((_BLK, _EMB), lambda i: (i, 0)),
            pl.BlockSpec((_EMB, _NEMB), lambda i: (0, 0)),
            pl.BlockSpec((_NEMB, _EMB), lambda i: (0, 0)),
            pl.BlockSpec((_NEMB, _EMB), lambda i: (0, 0)),
            pl.BlockSpec((_NEMB, _EMB), lambda i: (0, 0)),
            pl.BlockSpec((_BLK, 1), lambda i: (i, 0)),
            pl.BlockSpec((1, _NEMB), lambda i: (0, 0)),
        ],
        out_specs=[
            pl.BlockSpec((_BLK, 1), lambda i: (i, 0)),
            pl.BlockSpec((_BLK, _EMB), lambda i: (i, 0)),
        ],
        out_shape=[
            jax.ShapeDtypeStruct((tokens, 1), jnp.int32),
            jax.ShapeDtypeStruct((tokens, _EMB), jnp.float32),
        ],
        compiler_params=pltpu.CompilerParams(
            dimension_semantics=("parallel",)),
    )(flat, w2x, wt1, wt2, wt3, c, w2)
    return q.reshape(x.shape), idx.reshape(x.shape[:-1])
